# baseline (speedup 1.0000x reference)
import functools

import jax
import jax.numpy as jnp
from jax import lax
from jax.experimental import pallas as pl
from jax.experimental.pallas import tpu as pltpu
from jax.experimental.pallas import tpu_sc as plsc

B = 4096
F = 26
D = 64
LANES = 16
NUM_WORKERS = 32
ROWS_PER_W = B // NUM_WORKERS
CHUNK_ROWS = 16
NSUB = 4
SUB_ROWS = CHUNK_ROWS // NSUB
SUB_IDX = SUB_ROWS * F
IDX_PER_CHUNK = CHUNK_ROWS * F
NCHUNK = ROWS_PER_W // CHUNK_ROWS
NBUF = 2


def _emb_body(idx_hbm, table_hbm, out_hbm, idx_v, rows_v, out_v, sem0, sem1):
    sems = (sem0, sem1)
    cid = lax.axis_index("c")
    sid = lax.axis_index("s")
    wid = sid * 2 + cid
    obase = wid * ROWS_PER_W

    def load_idx(c, buf):
        row0 = obase + c * CHUNK_ROWS
        pltpu.sync_copy(idx_hbm.at[pl.ds(row0, CHUNK_ROWS)], idx_v.at[buf])

    def start_gathers(buf):
        for j in range(CHUNK_ROWS):
            pltpu.make_async_copy(
                table_hbm.at[idx_v.at[buf, j]],
                rows_v.at[buf, pl.ds(j * F, F)],
                sems[buf],
            ).start()

    def wait_gathers(buf):
        for j in range(CHUNK_ROWS):
            pltpu.make_async_copy(
                table_hbm.at[idx_v.at[buf, j]],
                rows_v.at[buf, pl.ds(j * F, F)],
                sems[buf],
            ).wait()

    def compute_store(c, buf):
        def row_body(i, carry):
            r0 = i * F
            for d in range(D // LANES):
                sl = pl.ds(d * LANES, LANES)
                acc = None
                for f in range(F):
                    v = rows_v[buf, r0 + f, sl]
                    acc = v if acc is None else acc + v
                out_v[i, sl] = acc
            return carry

        lax.fori_loop(0, CHUNK_ROWS, row_body, 0)
        orow = obase + c * CHUNK_ROWS
        pltpu.sync_copy(out_v, out_hbm.at[pl.ds(orow, CHUNK_ROWS)])

    load_idx(0, 0)
    start_gathers(0)

    def outer(it, carry):
        c2 = it * NBUF
        for b in range(NBUF):
            c = c2 + b
            nxt = c + 1

            @pl.when(nxt < NCHUNK)
            def _():
                load_idx(nxt, 1 - b)
                start_gathers(1 - b)

            wait_gathers(b)
            compute_store(c, b)
        return carry

    lax.fori_loop(0, NCHUNK // NBUF, outer, 0)


def kernel(x, table):
    mesh = plsc.VectorSubcoreMesh(core_axis_name="c", subcore_axis_name="s")
    k = functools.partial(
        pl.kernel,
        mesh=mesh,
        out_type=jax.ShapeDtypeStruct((B, D), jnp.float32),
        scratch_types=[
            pltpu.VMEM((NBUF, CHUNK_ROWS, F), jnp.int32),
            pltpu.VMEM((NBUF, IDX_PER_CHUNK, D), jnp.float32),
            pltpu.VMEM((CHUNK_ROWS, D), jnp.float32),
            pltpu.SemaphoreType.DMA,
            pltpu.SemaphoreType.DMA,
        ],
        compiler_params=pltpu.CompilerParams(use_tc_tiling_on_sc=False),
    )(_emb_body)
    return k(x, table)


# speedup vs baseline: 1.1020x; 1.1020x over previous
import functools

import jax
import jax.numpy as jnp
from jax import lax
from jax.experimental import pallas as pl
from jax.experimental.pallas import tpu as pltpu
from jax.experimental.pallas import tpu_sc as plsc

B = 4096
F = 26
D = 64
LANES = 16
NUM_WORKERS = 32
ROWS_PER_W = B // NUM_WORKERS
CHUNK_ROWS = 32
NCHUNK = ROWS_PER_W // CHUNK_ROWS
NBUF = 2


def _emb_body(idx_hbm, table_hbm, out_hbm, idx_v, rows_v, out_v, sem0, sem1):
    sems = (sem0, sem1)
    cid = lax.axis_index("c")
    sid = lax.axis_index("s")
    wid = sid * 2 + cid
    obase = wid * ROWS_PER_W

    def load_idx(c, buf):
        col0 = obase + c * CHUNK_ROWS
        pltpu.sync_copy(idx_hbm.at[:, pl.ds(col0, CHUNK_ROWS)], idx_v.at[buf])

    def start_gathers(buf):
        for f in range(F):
            pltpu.make_async_copy(
                table_hbm.at[idx_v.at[buf, f]], rows_v.at[buf, f], sems[buf]
            ).start()

    def wait_gathers(buf):
        for f in range(F):
            pltpu.make_async_copy(
                table_hbm.at[idx_v.at[buf, f]], rows_v.at[buf, f], sems[buf]
            ).wait()

    def compute_store(c, buf):
        def row_body(i, carry):
            for d in range(D // LANES):
                sl = pl.ds(d * LANES, LANES)
                acc = None
                for f in range(F):
                    v = rows_v[buf, f, i, sl]
                    acc = v if acc is None else acc + v
                out_v[i, sl] = acc
            return carry

        lax.fori_loop(0, CHUNK_ROWS, row_body, 0)
        orow = obase + c * CHUNK_ROWS
        pltpu.sync_copy(out_v, out_hbm.at[pl.ds(orow, CHUNK_ROWS)])

    load_idx(0, 0)
    start_gathers(0)

    def outer(it, carry):
        c2 = it * NBUF
        for b in range(NBUF):
            c = c2 + b
            nxt = c + 1

            @pl.when(nxt < NCHUNK)
            def _():
                load_idx(nxt, 1 - b)
                start_gathers(1 - b)

            wait_gathers(b)
            compute_store(c, b)
        return carry

    lax.fori_loop(0, NCHUNK // NBUF, outer, 0)


def kernel(x, table):
    xt = x.T
    mesh = plsc.VectorSubcoreMesh(core_axis_name="c", subcore_axis_name="s")
    k = functools.partial(
        pl.kernel,
        mesh=mesh,
        out_type=jax.ShapeDtypeStruct((B, D), jnp.float32),
        scratch_types=[
            pltpu.VMEM((NBUF, F, CHUNK_ROWS), jnp.int32),
            pltpu.VMEM((NBUF, F, CHUNK_ROWS, D), jnp.float32),
            pltpu.VMEM((CHUNK_ROWS, D), jnp.float32),
            pltpu.SemaphoreType.DMA,
            pltpu.SemaphoreType.DMA,
        ],
        compiler_params=pltpu.CompilerParams(use_tc_tiling_on_sc=False),
    )(_emb_body)
    return k(xt, table)
